# Initial kernel scaffold; baseline (speedup 1.0000x reference)
import functools

import numpy as np
import jax
import jax.numpy as jnp
from jax import lax
from jax.experimental import pallas as pl
from jax.experimental.pallas import tpu as pltpu
from jax.experimental.pallas import tpu_sc as plsc

N = 10000
E = 320000
H = 4
C = 32
HID = H * C

NC = 2
NS = 16
NW = NC * NS
EPW = E // NW
BE = 80
NB = EPW // BE
ACC_W = 144
RPS = N // NS

BN = 2000


def _dense_body(h_ref, wa_ref, ba_ref, qt_ref, kv_ref, r_ref):
    d = jnp.dot(h_ref[...], wa_ref[...], preferred_element_type=jnp.float32)
    d = d + ba_ref[...]
    qt_ref[...] = d[:, :HID]
    kv_ref[...] = d[:, HID:3 * HID]
    r_ref[...] = d[:, 3 * HID:]


def _dense_call(h, WA, bA):
    return pl.pallas_call(
        _dense_body,
        grid=(N // BN,),
        in_specs=[
            pl.BlockSpec((BN, HID), lambda i: (i, 0)),
            pl.BlockSpec((HID, 4 * HID), lambda i: (0, 0)),
            pl.BlockSpec((1, 4 * HID), lambda i: (0, 0)),
        ],
        out_specs=[
            pl.BlockSpec((BN, HID), lambda i: (i, 0)),
            pl.BlockSpec((BN, 2 * HID), lambda i: (i, 0)),
            pl.BlockSpec((BN, HID), lambda i: (i, 0)),
        ],
        out_shape=[
            jax.ShapeDtypeStruct((N, HID), jnp.float32),
            jax.ShapeDtypeStruct((N, 2 * HID), jnp.float32),
            jax.ShapeDtypeStruct((N, HID), jnp.float32),
        ],
    )(h, WA, bA)


def _combine_body(a0_ref, a1_ref, r_ref, wa_ref, ba_ref, wb13_ref, wb23_ref,
                  exp_ref, h_ref, qt_ref, kv_ref, rn_ref):
    num = a0_ref[:, :HID] + a1_ref[:, :HID]
    tail = a0_ref[:, HID:] + a1_ref[:, HID:]
    denf = jnp.dot(tail, exp_ref[...], preferred_element_type=jnp.float32)
    out = num / (denf + 1e-16)
    r = r_ref[...]
    g = jax.nn.sigmoid(
        jnp.dot(out, wb13_ref[...], preferred_element_type=jnp.float32)
        + jnp.dot(r, wb23_ref[...], preferred_element_type=jnp.float32))
    h = g * r + (1.0 - g) * out
    h_ref[...] = h
    d = jnp.dot(h, wa_ref[...], preferred_element_type=jnp.float32)
    d = d + ba_ref[...]
    qt_ref[...] = d[:, :HID]
    kv_ref[...] = d[:, HID:3 * HID]
    rn_ref[...] = d[:, 3 * HID:]


def _combine_call(acc0, acc1, r, WA, bA, Wb13, Wb23, EXPAND):
    return pl.pallas_call(
        _combine_body,
        grid=(N // BN,),
        in_specs=[
            pl.BlockSpec((BN, ACC_W), lambda i: (i, 0)),
            pl.BlockSpec((BN, ACC_W), lambda i: (i, 0)),
            pl.BlockSpec((BN, HID), lambda i: (i, 0)),
            pl.BlockSpec((HID, 4 * HID), lambda i: (0, 0)),
            pl.BlockSpec((1, 4 * HID), lambda i: (0, 0)),
            pl.BlockSpec((HID, HID), lambda i: (0, 0)),
            pl.BlockSpec((HID, HID), lambda i: (0, 0)),
            pl.BlockSpec((16, HID), lambda i: (0, 0)),
        ],
        out_specs=[
            pl.BlockSpec((BN, HID), lambda i: (i, 0)),
            pl.BlockSpec((BN, HID), lambda i: (i, 0)),
            pl.BlockSpec((BN, 2 * HID), lambda i: (i, 0)),
            pl.BlockSpec((BN, HID), lambda i: (i, 0)),
        ],
        out_shape=[
            jax.ShapeDtypeStruct((N, HID), jnp.float32),
            jax.ShapeDtypeStruct((N, HID), jnp.float32),
            jax.ShapeDtypeStruct((N, 2 * HID), jnp.float32),
            jax.ShapeDtypeStruct((N, HID), jnp.float32),
        ],
    )(acc0, acc1, r, WA, bA, Wb13, Wb23, EXPAND)


def _edge_body(qt_hbm, kv_hbm, we_hbm, src_hbm, dst_hbm, w_hbm, z_hbm,
               acc_hbm, srcb, dstb, wb, kvb, qb, ob, web, accs):
    cid = lax.axis_index("c")
    sid = lax.axis_index("s")
    wid = cid * NS + sid

    pltpu.sync_copy(z_hbm.at[pl.ds(sid * RPS, RPS)],
                    accs.at[pl.ds(sid * RPS, RPS)])
    pltpu.sync_copy(we_hbm, web)
    plsc.subcore_barrier()

    we = [web[0, pl.ds(16 * j, 16)] for j in range(8)]

    @pl.loop(0, NB)
    def _blk(j):
        base = wid * EPW + j * BE
        pltpu.sync_copy(src_hbm.at[pl.ds(base, BE)], srcb)
        pltpu.sync_copy(dst_hbm.at[pl.ds(base, BE)], dstb)
        pltpu.sync_copy(w_hbm.at[pl.ds(base, BE)], wb)
        pltpu.sync_copy(kv_hbm.at[srcb], kvb)
        pltpu.sync_copy(qt_hbm.at[dstb], qb)

        @pl.loop(0, BE)
        def _edge(e):
            wv = jnp.full((16,), wb[e], jnp.float32)
            lane = lax.iota(jnp.int32, 16)
            pbs = []
            dv = jnp.zeros((16,), jnp.float32)
            for hh in range(H):
                s0 = qb[e, pl.ds(32 * hh, 16)] * (
                    kvb[e, pl.ds(32 * hh, 16)] + wv * we[2 * hh])
                s1 = qb[e, pl.ds(32 * hh + 16, 16)] * (
                    kvb[e, pl.ds(32 * hh + 16, 16)] + wv * we[2 * hh + 1])
                dot = jnp.sum(s0 + s1)
                pb = jnp.exp(jnp.full((16,), dot, jnp.float32))
                pbs.append(pb)
                dv = jnp.where(lane == hh, pb, dv)
            for hh in range(H):
                v0 = kvb[e, pl.ds(HID + 32 * hh, 16)] + wv * we[2 * hh]
                v1 = kvb[e, pl.ds(HID + 32 * hh + 16, 16)] + wv * we[2 * hh + 1]
                ob[e, pl.ds(32 * hh, 16)] = pbs[hh] * v0
                ob[e, pl.ds(32 * hh + 16, 16)] = pbs[hh] * v1
            ob[e, pl.ds(HID, 16)] = dv

        pltpu.sync_copy(ob, accs.at[dstb], add=True)

    plsc.subcore_barrier()
    pltpu.sync_copy(accs.at[pl.ds(sid * RPS, RPS)],
                    acc_hbm.at[cid, pl.ds(sid * RPS, RPS)])


def _edge_call(qt, kv, We, src, dst, w, Z):
    mesh = plsc.VectorSubcoreMesh(core_axis_name="c", subcore_axis_name="s")
    kern = pl.kernel(
        _edge_body,
        out_type=jax.ShapeDtypeStruct((NC, N, ACC_W), jnp.float32),
        mesh=mesh,
        scratch_types=[
            pltpu.VMEM((BE,), jnp.int32),
            pltpu.VMEM((BE,), jnp.int32),
            pltpu.VMEM((BE,), jnp.float32),
            pltpu.VMEM((BE, 2 * HID), jnp.float32),
            pltpu.VMEM((BE, HID), jnp.float32),
            pltpu.VMEM((BE, ACC_W), jnp.float32),
            pltpu.VMEM((1, HID), jnp.float32),
            pltpu.VMEM_SHARED((N, ACC_W), jnp.float32),
        ],
    )
    return kern(qt, kv, We, src, dst, w, Z)


def _pack_params(p, s, sc):
    WA = jnp.concatenate(
        [p["Wq" + s] * sc, p["Wk" + s], p["Wv" + s], p["Ws" + s]], axis=1)
    bA = jnp.concatenate(
        [p["bq" + s] * sc, p["bk" + s], p["bv" + s], p["bs" + s]])[None, :]
    Wb = p["Wb" + s]
    Wb13 = jnp.tile(Wb[:HID] + Wb[2 * HID:], (1, HID))
    Wb23 = jnp.tile(Wb[HID:2 * HID] - Wb[2 * HID:], (1, HID))
    return WA, bA, Wb13, Wb23, p["We" + s]


def kernel(x, edge_index, edge_weight, params):
    src = edge_index[0]
    dst = edge_index[1]
    w = edge_weight
    sc = np.float32(1.0 / np.sqrt(C))

    WA1, bA1, Wb13_1, Wb23_1, We1 = _pack_params(params, "1", sc)
    WA2, bA2, Wb13_2, Wb23_2, We2 = _pack_params(params, "2", sc)

    expand = np.zeros((16, HID), np.float32)
    for hh in range(H):
        expand[hh, hh * C:(hh + 1) * C] = 1.0
    EXPAND = jnp.asarray(expand)
    Z = jnp.zeros((N, ACC_W), jnp.float32)

    qt, kv, r = _dense_call(x, WA1, bA1)
    h = x
    for layer in range(8):
        if layer == 0:
            We_l, Wb13, Wb23 = We1, Wb13_1, Wb23_1
        else:
            We_l, Wb13, Wb23 = We2, Wb13_2, Wb23_2
        acc = _edge_call(qt, kv, We_l, src, dst, w, Z)
        h, qt, kv, r = _combine_call(acc[0], acc[1], r, WA2, bA2, Wb13, Wb23,
                                     EXPAND)
    return h


# speedup vs baseline: 19.1636x; 19.1636x over previous
import functools

import numpy as np
import jax
import jax.numpy as jnp
from jax import lax
from jax.experimental import pallas as pl
from jax.experimental.pallas import tpu as pltpu
from jax.experimental.pallas import tpu_sc as plsc

N = 10000
E = 320000
H = 4
C = 32
HID = H * C

NC = 2
NS = 16
NW = NC * NS
EPW = E // NW
BE = 40
NB = EPW // BE
NP = 10112
RPS = NP // NS
ND = 1280
DPS = ND // NS

BN = 2000


def _dense_body(h_ref, wa_ref, ba_ref, qt_ref, kv_ref, r_ref):
    d = jnp.dot(h_ref[...], wa_ref[...], preferred_element_type=jnp.float32)
    d = d + ba_ref[...]
    qt_ref[...] = d[:, :HID]
    kv_ref[...] = d[:, HID:3 * HID]
    r_ref[...] = d[:, 3 * HID:]


def _dense_call(h, WA, bA):
    return pl.pallas_call(
        _dense_body,
        grid=(N // BN,),
        in_specs=[
            pl.BlockSpec((BN, HID), lambda i: (i, 0)),
            pl.BlockSpec((HID, 4 * HID), lambda i: (0, 0)),
            pl.BlockSpec((1, 4 * HID), lambda i: (0, 0)),
        ],
        out_specs=[
            pl.BlockSpec((BN, HID), lambda i: (i, 0)),
            pl.BlockSpec((BN, 2 * HID), lambda i: (i, 0)),
            pl.BlockSpec((BN, HID), lambda i: (i, 0)),
        ],
        out_shape=[
            jax.ShapeDtypeStruct((N, HID), jnp.float32),
            jax.ShapeDtypeStruct((N, 2 * HID), jnp.float32),
            jax.ShapeDtypeStruct((N, HID), jnp.float32),
        ],
    )(h, WA, bA)


def _combine_body(a0_ref, a1_ref, d0_ref, d1_ref, r_ref, wa_ref, ba_ref,
                  wb13_ref, wb23_ref, exp_ref, h_ref, qt_ref, kv_ref, rn_ref):
    num = a0_ref[...] + a1_ref[...]
    tail = d0_ref[...] + d1_ref[...]
    denf = jnp.dot(tail, exp_ref[...], preferred_element_type=jnp.float32)
    out = num / (denf + 1e-16)
    r = r_ref[...]
    g = jax.nn.sigmoid(
        jnp.dot(out, wb13_ref[...], preferred_element_type=jnp.float32)
        + jnp.dot(r, wb23_ref[...], preferred_element_type=jnp.float32))
    h = g * r + (1.0 - g) * out
    h_ref[...] = h
    d = jnp.dot(h, wa_ref[...], preferred_element_type=jnp.float32)
    d = d + ba_ref[...]
    qt_ref[...] = d[:, :HID]
    kv_ref[...] = d[:, HID:3 * HID]
    rn_ref[...] = d[:, 3 * HID:]


def _combine_call(acc0, acc1, d0, d1, r, WA, bA, Wb13, Wb23, EXPAND):
    return pl.pallas_call(
        _combine_body,
        grid=(N // BN,),
        in_specs=[
            pl.BlockSpec((BN, HID), lambda i: (i, 0)),
            pl.BlockSpec((BN, HID), lambda i: (i, 0)),
            pl.BlockSpec((BN, 16), lambda i: (i, 0)),
            pl.BlockSpec((BN, 16), lambda i: (i, 0)),
            pl.BlockSpec((BN, HID), lambda i: (i, 0)),
            pl.BlockSpec((HID, 4 * HID), lambda i: (0, 0)),
            pl.BlockSpec((1, 4 * HID), lambda i: (0, 0)),
            pl.BlockSpec((HID, HID), lambda i: (0, 0)),
            pl.BlockSpec((HID, HID), lambda i: (0, 0)),
            pl.BlockSpec((16, HID), lambda i: (0, 0)),
        ],
        out_specs=[
            pl.BlockSpec((BN, HID), lambda i: (i, 0)),
            pl.BlockSpec((BN, HID), lambda i: (i, 0)),
            pl.BlockSpec((BN, 2 * HID), lambda i: (i, 0)),
            pl.BlockSpec((BN, HID), lambda i: (i, 0)),
        ],
        out_shape=[
            jax.ShapeDtypeStruct((N, HID), jnp.float32),
            jax.ShapeDtypeStruct((N, HID), jnp.float32),
            jax.ShapeDtypeStruct((N, 2 * HID), jnp.float32),
            jax.ShapeDtypeStruct((N, HID), jnp.float32),
        ],
    )(acc0, acc1, d0, d1, r, WA, bA, Wb13, Wb23, EXPAND)


def _edge_compute(qb, kvb, ob, ob2, we, e, w_scalar, d_scalar):
    wv = jnp.full((16,), w_scalar, jnp.float32)
    lane = lax.iota(jnp.int32, 16)
    pbs = []
    dv = jnp.zeros((16,), jnp.float32)
    for hh in range(H):
        s0 = qb[e, pl.ds(32 * hh, 16)] * (
            kvb[e, pl.ds(32 * hh, 16)] + wv * we[2 * hh])
        s1 = qb[e, pl.ds(32 * hh + 16, 16)] * (
            kvb[e, pl.ds(32 * hh + 16, 16)] + wv * we[2 * hh + 1])
        dot = jnp.sum(s0 + s1)
        pb = jnp.exp(jnp.full((16,), dot, jnp.float32))
        pbs.append(pb)
        dv = jnp.where(lane == hh, pb, dv)
    for hh in range(H):
        v0 = kvb[e, pl.ds(HID + 32 * hh, 16)] + wv * we[2 * hh]
        v1 = kvb[e, pl.ds(HID + 32 * hh + 16, 16)] + wv * we[2 * hh + 1]
        ob[e, pl.ds(32 * hh, 16)] = pbs[hh] * v0
        ob[e, pl.ds(32 * hh + 16, 16)] = pbs[hh] * v1
    zv = jnp.zeros((16,), jnp.float32)
    r8 = jnp.full((16,), (d_scalar & 7), jnp.int32)
    for m in range(8):
        ob2[e, pl.ds(16 * m, 16)] = jnp.where(r8 == m, dv, zv)


def _edge_body(qt_hbm, kv_hbm, we_hbm, src_hbm, dst_hbm, dst8_hbm, w_hbm,
               z_hbm, acc_hbm, den_hbm, srcb, dstb, dstw, dst8b, wb, kvb, qb,
               ob, ob2, web, accs, dens):
    cid = lax.axis_index("c")
    sid = lax.axis_index("s")
    wid = cid * NS + sid

    pltpu.sync_copy(z_hbm.at[pl.ds(sid * RPS, RPS)],
                    accs.at[pl.ds(sid * RPS, RPS)])
    pltpu.sync_copy(z_hbm.at[pl.ds(sid * DPS, DPS)],
                    dens.at[pl.ds(sid * DPS, DPS)])
    pltpu.sync_copy(we_hbm, web)
    plsc.subcore_barrier()

    we = [web[0, pl.ds(16 * j, 16)] for j in range(8)]

    @pl.loop(0, NB)
    def _blk(j):
        base = wid * EPW + j * BE
        pltpu.sync_copy(src_hbm.at[pl.ds(base, BE)], srcb)
        pltpu.sync_copy(dst_hbm.at[pl.ds(base, BE)], dstb)
        pltpu.sync_copy(dst_hbm.at[pl.ds(base, BE)], dstw.at[0])
        pltpu.sync_copy(dst8_hbm.at[pl.ds(base, BE)], dst8b.at[0])
        pltpu.sync_copy(w_hbm.at[pl.ds(base, BE)], wb)
        pltpu.sync_copy(kv_hbm.at[srcb], kvb)
        pltpu.sync_copy(qt_hbm.at[dstb], qb)

        @pl.loop(0, BE // 16)
        def _grp(g):
            wgv = wb[pl.ds(16 * g, 16)]
            dgv = dstb[pl.ds(16 * g, 16)]
            for i in range(16):
                _edge_compute(qb, kvb, ob, ob2, we, 16 * g + i, wgv[i], dgv[i])

        pltpu.sync_copy(ob, accs.at[dstw.at[0]], add=True)
        pltpu.sync_copy(ob2, dens.at[dst8b.at[0]], add=True)

    plsc.subcore_barrier()
    pltpu.sync_copy(accs.at[pl.ds(sid * RPS, RPS)],
                    acc_hbm.at[cid, pl.ds(sid * RPS, RPS)])
    pltpu.sync_copy(dens.at[pl.ds(sid * DPS, DPS)],
                    den_hbm.at[cid, pl.ds(sid * DPS, DPS)])


def _sc_compiler_params():
    import dataclasses
    cp = pltpu.CompilerParams()
    if "needs_layout_passes" in pltpu.CompilerParams.__dataclass_fields__:
        cp = dataclasses.replace(cp, needs_layout_passes=False)
    return cp


def _edge_call(qt, kv, We, src, dst, dst8, w, Z):
    mesh = plsc.VectorSubcoreMesh(core_axis_name="c", subcore_axis_name="s")
    kern = pl.kernel(
        _edge_body,
        out_type=[
            jax.ShapeDtypeStruct((NC, NP, HID), jnp.float32),
            jax.ShapeDtypeStruct((NC, ND, HID), jnp.float32),
        ],
        mesh=mesh,
        compiler_params=_sc_compiler_params(),
        scratch_types=[
            pltpu.VMEM((BE,), jnp.int32),
            pltpu.VMEM((BE,), jnp.int32),
            pltpu.VMEM((1, BE), jnp.int32),
            pltpu.VMEM((1, BE), jnp.int32),
            pltpu.VMEM((BE,), jnp.float32),
            pltpu.VMEM((BE, 2 * HID), jnp.float32),
            pltpu.VMEM((BE, HID), jnp.float32),
            pltpu.VMEM((BE, HID), jnp.float32),
            pltpu.VMEM((BE, HID), jnp.float32),
            pltpu.VMEM((1, HID), jnp.float32),
            pltpu.VMEM_SHARED((NP, HID), jnp.float32),
            pltpu.VMEM_SHARED((ND, HID), jnp.float32),
        ],
    )
    return kern(qt, kv, We, src, dst, dst8, w, Z)


def _pack_params(p, s, sc):
    WA = jnp.concatenate(
        [p["Wq" + s] * sc, p["Wk" + s], p["Wv" + s], p["Ws" + s]], axis=1)
    bA = jnp.concatenate(
        [p["bq" + s] * sc, p["bk" + s], p["bv" + s], p["bs" + s]])[None, :]
    Wb = p["Wb" + s]
    Wb13 = jnp.tile(Wb[:HID] + Wb[2 * HID:], (1, HID))
    Wb23 = jnp.tile(Wb[HID:2 * HID] - Wb[2 * HID:], (1, HID))
    return WA, bA, Wb13, Wb23, p["We" + s]


def kernel(x, edge_index, edge_weight, params):
    src = edge_index[0]
    dst = edge_index[1]
    w = edge_weight
    sc = np.float32(1.0 / np.sqrt(C))

    WA1, bA1, Wb13_1, Wb23_1, We1 = _pack_params(params, "1", sc)
    WA2, bA2, Wb13_2, Wb23_2, We2 = _pack_params(params, "2", sc)

    expand = np.zeros((16, HID), np.float32)
    for hh in range(H):
        expand[hh, hh * C:(hh + 1) * C] = 1.0
    EXPAND = jnp.asarray(expand)
    Z = jnp.zeros((NP, HID), jnp.float32)
    dst8 = lax.shift_right_logical(dst, 3)

    qt, kv, r = _dense_call(x, WA1, bA1)
    h = x
    for layer in range(8):
        if layer == 0:
            We_l, Wb13, Wb23 = We1, Wb13_1, Wb23_1
        else:
            We_l, Wb13, Wb23 = We2, Wb13_2, Wb23_2
        acc, den = _edge_call(qt, kv, We_l, src, dst, dst8, w, Z)
        d0 = den[0].reshape(ND * 8, 16)
        d1 = den[1].reshape(ND * 8, 16)
        h, qt, kv, r = _combine_call(acc[0], acc[1], d0, d1, r, WA2, bA2,
                                     Wb13, Wb23, EXPAND)
    return h
